# Initial kernel scaffold; baseline (speedup 1.0000x reference)
import jax
import jax.numpy as jnp
from jax.experimental import pallas as pl


def kernel(x, edge_index, W1, b1, Wmu, bmu, Wls, bls):
    raise NotImplementedError("write your pallas kernel here")


# speedup vs baseline: 14.9228x; 14.9228x over previous
import functools

import jax
import jax.numpy as jnp
from jax import lax
from jax.experimental import pallas as pl
from jax.experimental.pallas import tpu as pltpu
from jax.experimental.pallas import tpu_sc as plsc

N = 10000
NPAD = 10240
E = 320000
EPAD = 323584
NC, NS = 2, 16
BLK = 128
NB_AGG = EPAD // (NS * BLK)
NB_DEG = EPAD // (NC * NS * BLK)
SEG = NPAD // NS
D = 64
DEG_D = 8
RB = 512
GRID = NPAD // RB

_mesh = plsc.VectorSubcoreMesh(
    core_axis_name="c", subcore_axis_name="s", num_cores=NC, num_subcores=NS)
_sc_params = pltpu.CompilerParams(use_tc_tiling_on_sc=False)


def _deg_body(dst_ref, ones_ref, zeros_ref, out_ref, idx_v, ones_v, wb_v, acc_spm):
    c = lax.axis_index("c")
    s = lax.axis_index("s")
    w = c * NS + s
    seg = pl.ds(s * SEG, SEG)
    pltpu.sync_copy(ones_ref, ones_v)
    pltpu.sync_copy(zeros_ref, wb_v)
    pltpu.sync_copy(wb_v, acc_spm.at[seg])
    pltpu.sync_copy(dst_ref.at[w], idx_v)
    plsc.subcore_barrier()

    def body(b, carry):
        pltpu.sync_copy(ones_v, acc_spm.at[idx_v.at[b]], add=True)
        return carry

    lax.fori_loop(0, NB_DEG, body, 0)
    plsc.subcore_barrier()
    pltpu.sync_copy(acc_spm.at[seg], wb_v)
    pltpu.sync_copy(wb_v, out_ref.at[c, seg])


_deg = pl.kernel(
    _deg_body,
    out_type=jax.ShapeDtypeStruct((NC, NPAD, DEG_D), jnp.float32),
    mesh=_mesh,
    scratch_types=[
        pltpu.VMEM((NB_DEG, BLK), jnp.int32),
        pltpu.VMEM((BLK, DEG_D), jnp.float32),
        pltpu.VMEM((SEG, DEG_D), jnp.float32),
        pltpu.VMEM_SHARED((NPAD, DEG_D), jnp.float32),
    ],
    compiler_params=_sc_params,
)


def _agg_body(src_ref, dst_ref, tbl_ref, zeros_ref, out_ref,
              src_v, dst_v, rows_v, wb_v, acc_spm, sem):
    c = lax.axis_index("c")
    s = lax.axis_index("s")
    seg = pl.ds(s * SEG, SEG)
    pltpu.sync_copy(zeros_ref, wb_v)
    pltpu.sync_copy(wb_v, acc_spm.at[seg])
    pltpu.sync_copy(src_ref.at[s], src_v)
    pltpu.sync_copy(dst_ref.at[s], dst_v)
    plsc.subcore_barrier()

    def body(b, carry):
        pltpu.async_copy(tbl_ref.at[c].at[src_v.at[b]], rows_v, sem).wait()
        pltpu.sync_copy(rows_v, acc_spm.at[dst_v.at[b]], add=True)
        return carry

    lax.fori_loop(0, NB_AGG, body, 0)
    plsc.subcore_barrier()
    pltpu.sync_copy(acc_spm.at[seg], wb_v)
    pltpu.sync_copy(wb_v, out_ref.at[c, seg])


_agg = pl.kernel(
    _agg_body,
    out_type=jax.ShapeDtypeStruct((NC, NPAD, D), jnp.float32),
    mesh=_mesh,
    scratch_types=[
        pltpu.VMEM((NB_AGG, BLK), jnp.int32),
        pltpu.VMEM((NB_AGG, BLK), jnp.int32),
        pltpu.VMEM((BLK, D), jnp.float32),
        pltpu.VMEM((SEG, D), jnp.float32),
        pltpu.VMEM_SHARED((NPAD, D), jnp.float32),
        pltpu.SemaphoreType.DMA,
    ],
    compiler_params=_sc_params,
)


def _dis_col(degp):
    return lax.rsqrt(degp[0, :, 0:1] + degp[1, :, 0:1] + 1.0)


def _tc1_body(x_ref, w1_ref, degp_ref, tbl_ref):
    dis = _dis_col(degp_ref[...])
    h = jnp.dot(x_ref[...], w1_ref[...], preferred_element_type=jnp.float32)
    t = h * dis
    tbl_ref[0] = t[:, :D]
    tbl_ref[1] = t[:, D:]


_tc1 = pl.pallas_call(
    _tc1_body,
    grid=(GRID,),
    in_specs=[
        pl.BlockSpec((RB, 128), lambda i: (i, 0)),
        pl.BlockSpec((128, 128), lambda i: (0, 0)),
        pl.BlockSpec((NC, RB, DEG_D), lambda i: (0, i, 0)),
    ],
    out_specs=pl.BlockSpec((NC, RB, D), lambda i: (0, i, 0)),
    out_shape=jax.ShapeDtypeStruct((NC, NPAD, D), jnp.float32),
)


def _tc2_body(acc_ref, tbl_ref, degp_ref, wcat_ref, b1_ref, out_ref):
    dis = _dis_col(degp_ref[...])
    h0 = dis * (acc_ref[0] + tbl_ref[0])
    h1 = dis * (acc_ref[1] + tbl_ref[1])
    hcat = jnp.concatenate([h0, h1], axis=1) + b1_ref[...]
    hr = jnp.maximum(hcat, 0.0)
    h2 = jnp.dot(hr, wcat_ref[...], preferred_element_type=jnp.float32)
    t2 = h2 * dis
    out_ref[0] = t2[:, :D]
    out_ref[1] = t2[:, D:]


_tc2 = pl.pallas_call(
    _tc2_body,
    grid=(GRID,),
    in_specs=[
        pl.BlockSpec((NC, RB, D), lambda i: (0, i, 0)),
        pl.BlockSpec((NC, RB, D), lambda i: (0, i, 0)),
        pl.BlockSpec((NC, RB, DEG_D), lambda i: (0, i, 0)),
        pl.BlockSpec((128, 128), lambda i: (0, 0)),
        pl.BlockSpec((1, 128), lambda i: (0, 0)),
    ],
    out_specs=pl.BlockSpec((NC, RB, D), lambda i: (0, i, 0)),
    out_shape=jax.ShapeDtypeStruct((NC, NPAD, D), jnp.float32),
)


def _tc3_body(acc_ref, tbl_ref, degp_ref, bmu_ref, bls_ref, mu_ref, ls_ref):
    dis = _dis_col(degp_ref[...])
    mu_ref[...] = dis * (acc_ref[0] + tbl_ref[0]) + bmu_ref[...]
    ls_ref[...] = dis * (acc_ref[1] + tbl_ref[1]) + bls_ref[...]


_tc3 = pl.pallas_call(
    _tc3_body,
    grid=(GRID,),
    in_specs=[
        pl.BlockSpec((NC, RB, D), lambda i: (0, i, 0)),
        pl.BlockSpec((NC, RB, D), lambda i: (0, i, 0)),
        pl.BlockSpec((NC, RB, DEG_D), lambda i: (0, i, 0)),
        pl.BlockSpec((1, D), lambda i: (0, 0)),
        pl.BlockSpec((1, D), lambda i: (0, 0)),
    ],
    out_specs=[
        pl.BlockSpec((RB, D), lambda i: (i, 0)),
        pl.BlockSpec((RB, D), lambda i: (i, 0)),
    ],
    out_shape=[
        jax.ShapeDtypeStruct((NPAD, D), jnp.float32),
        jax.ShapeDtypeStruct((NPAD, D), jnp.float32),
    ],
)


def kernel(x, edge_index, W1, b1, Wmu, bmu, Wls, bls):
    f32 = jnp.float32
    src = edge_index[0].astype(jnp.int32)
    dst = edge_index[1].astype(jnp.int32)
    pad = jnp.full((EPAD - E,), N, jnp.int32)
    srcp = jnp.concatenate([src, pad])
    dstp = jnp.concatenate([dst, pad])
    src_a = srcp.reshape(NS, NB_AGG, BLK)
    dst_a = dstp.reshape(NS, NB_AGG, BLK)
    dst_d = dstp.reshape(NC * NS, NB_DEG, BLK)
    xp = jnp.pad(x, ((0, NPAD - N), (0, 0)))
    ones_in = jnp.ones((BLK, DEG_D), f32)
    zeros_deg = jnp.zeros((SEG, DEG_D), f32)
    zeros_agg = jnp.zeros((SEG, D), f32)
    wcat = jnp.concatenate([Wmu, Wls], axis=1)

    degp = _deg(dst_d, ones_in, zeros_deg)
    tbl1 = _tc1(xp, W1, degp)
    acc1 = _agg(src_a, dst_a, tbl1, zeros_agg)
    tbl2 = _tc2(acc1, tbl1, degp, wcat, b1.reshape(1, 128))
    acc2 = _agg(src_a, dst_a, tbl2, zeros_agg)
    mu, ls = _tc3(acc2, tbl2, degp, bmu.reshape(1, D), bls.reshape(1, D))
    return mu[:N], ls[:N]
